# baseline (speedup 1.0000x reference)
import functools

import jax
import jax.numpy as jnp
from jax import lax
from jax.experimental import pallas as pl
from jax.experimental.pallas import tpu as pltpu
from jax.experimental.pallas import tpu_sc as plsc

_K = 64
_N = 50000
_D = 256
_T = 80
_NCHUNK = _N // _T
_NW = 32
_SLOTS = -(-_NCHUNK // _NW)
_UNROLL = 32

_mesh = plsc.VectorSubcoreMesh(core_axis_name="c", subcore_axis_name="s")


@functools.partial(
    pl.kernel,
    out_type=jax.ShapeDtypeStruct((_NW, 16), jnp.float32),
    mesh=_mesh,
    scratch_types=[
        pltpu.VMEM((_T * _D,), jnp.float32),
        pltpu.VMEM((_T * _D,), jnp.float32),
        pltpu.VMEM((_T,), jnp.int32),
        pltpu.VMEM((_T,), jnp.int32),
        pltpu.VMEM((_K * _D,), jnp.float32),
        pltpu.VMEM((16,), jnp.float32),
        pltpu.SemaphoreType.DMA,
        pltpu.SemaphoreType.DMA,
        pltpu.SemaphoreType.DMA,
        pltpu.SemaphoreType.DMA,
    ],
    compiler_params=pltpu.CompilerParams(
        use_tc_tiling_on_sc=False, needs_layout_passes=False),
)
def _sc_loss(x_hbm, a_hbm, c_hbm, out_hbm,
             x_v0, x_v1, a_v0, a_v1, c_v, p_v,
             sx0, sx1, sa0, sa1):
    wid = lax.axis_index("s") * 2 + lax.axis_index("c")
    pltpu.sync_copy(c_hbm, c_v)

    lanes = lax.broadcasted_iota(jnp.int32, (16,), 0)
    ones = jnp.ones((16,), jnp.int32)
    dmask = jnp.full((16,), _D - 1, jnp.int32)
    fzeros = jnp.zeros((16,), jnp.float32)
    bufs = ((x_v0, a_v0, sx0, sa0), (x_v1, a_v1, sx1, sa1))

    def start(slot, buf):
        x_v, a_v, sx, sa = buf
        off = jnp.minimum(wid + slot * _NW, _NCHUNK - 1) * _T
        pltpu.make_async_copy(
            x_hbm.at[pl.ds(off * _D, _T * _D)], x_v, sx).start()
        pltpu.make_async_copy(a_hbm.at[pl.ds(off, _T)], a_v, sa).start()

    def process(slot, buf, acc):
        x_v, a_v, sx, sa = buf
        pltpu.make_async_copy(
            x_hbm.at[pl.ds(0, _T * _D)], x_v, sx).wait()
        pltpu.make_async_copy(a_hbm.at[pl.ds(0, _T)], a_v, sa).wait()
        valid = (wid + slot * _NW) < _NCHUNK
        vmask = jnp.where(valid, jnp.ones((16,), jnp.float32), fzeros)
        def group_body(g, part):
            va = a_v[pl.ds(g * 16, 16)]
            for l in range(16):
                p = g * 16 + l
                xoff = p * _D
                coff = va[l] * _D
                pa, pb = fzeros, fzeros
                for k in range(_D // 16):
                    vx = x_v[pl.ds(xoff + k * 16, 16)]
                    vc = c_v[pl.ds(coff + k * 16, 16)]
                    diff = vx - vc
                    if k % 2 == 0:
                        pa = pa + diff * diff
                    else:
                        pb = pb + diff * diff
                part = part + (pa + pb)
            return part

        part = lax.fori_loop(0, _T // 16, group_body, fzeros)
        acc = acc + vmask * part
        return acc

    start(0, bufs[0])

    def slot_pair(t, acc):
        s0 = t * 2
        start(s0 + 1, bufs[1])
        acc = process(s0, bufs[0], acc)
        start(s0 + 2, bufs[0])
        acc = process(s0 + 1, bufs[1], acc)
        return acc

    acc = lax.fori_loop(0, _SLOTS // 2, slot_pair, fzeros)
    pltpu.make_async_copy(x_hbm.at[pl.ds(0, _T * _D)], x_v0, sx0).wait()
    pltpu.make_async_copy(a_hbm.at[pl.ds(0, _T)], a_v0, sa0).wait()

    p_v[...] = acc
    pltpu.sync_copy(p_v, out_hbm.at[wid])


def kernel(x, cluster_assignments, cluster_centers):
    partials = _sc_loss(x.reshape(-1), cluster_assignments,
                        cluster_centers.reshape(-1))
    return jnp.sum(partials)


# speedup vs baseline: 1.9472x; 1.9472x over previous
import functools

import jax
import jax.numpy as jnp
from jax import lax
from jax.experimental import pallas as pl
from jax.experimental.pallas import tpu as pltpu
from jax.experimental.pallas import tpu_sc as plsc

_K = 64
_N = 50000
_D = 256
_T = 80
_NW = 32
_N_SC = 10000
_N_TC = _N - _N_SC
_NCHUNK = _N_SC // _T
_SLOTS = -(-_NCHUNK // _NW)
_UNROLL = 32
_B_TC = 10000
_NB_TC = _N_TC // _B_TC

_mesh = plsc.VectorSubcoreMesh(core_axis_name="c", subcore_axis_name="s")


@functools.partial(
    pl.kernel,
    out_type=jax.ShapeDtypeStruct((_NW, 16), jnp.float32),
    mesh=_mesh,
    scratch_types=[
        pltpu.VMEM((_T * _D,), jnp.float32),
        pltpu.VMEM((_T * _D,), jnp.float32),
        pltpu.VMEM((_T,), jnp.int32),
        pltpu.VMEM((_T,), jnp.int32),
        pltpu.VMEM((_K * _D,), jnp.float32),
        pltpu.VMEM((16,), jnp.float32),
        pltpu.SemaphoreType.DMA,
        pltpu.SemaphoreType.DMA,
        pltpu.SemaphoreType.DMA,
        pltpu.SemaphoreType.DMA,
    ],
    compiler_params=pltpu.CompilerParams(
        use_tc_tiling_on_sc=False, needs_layout_passes=False),
)
def _sc_loss(x_hbm, a_hbm, c_hbm, out_hbm,
             x_v0, x_v1, a_v0, a_v1, c_v, p_v,
             sx0, sx1, sa0, sa1):
    wid = lax.axis_index("s") * 2 + lax.axis_index("c")
    pltpu.sync_copy(c_hbm, c_v)

    lanes = lax.broadcasted_iota(jnp.int32, (16,), 0)
    ones = jnp.ones((16,), jnp.int32)
    dmask = jnp.full((16,), _D - 1, jnp.int32)
    fzeros = jnp.zeros((16,), jnp.float32)
    bufs = ((x_v0, a_v0, sx0, sa0), (x_v1, a_v1, sx1, sa1))

    def start(slot, buf):
        x_v, a_v, sx, sa = buf
        cid = jnp.minimum(wid + slot * _NW, _NCHUNK - 1)
        off = _N_TC + cid * _T
        pltpu.make_async_copy(
            x_hbm.at[pl.ds(off * _D, _T * _D)], x_v, sx).start()
        pltpu.make_async_copy(a_hbm.at[pl.ds(off, _T)], a_v, sa).start()

    def process(slot, buf, acc):
        x_v, a_v, sx, sa = buf
        pltpu.make_async_copy(
            x_hbm.at[pl.ds(0, _T * _D)], x_v, sx).wait()
        pltpu.make_async_copy(a_hbm.at[pl.ds(0, _T)], a_v, sa).wait()
        valid = (wid + slot * _NW) < _NCHUNK
        vmask = jnp.where(valid, jnp.ones((16,), jnp.float32), fzeros)
        for g in range(_T // 16):
            xbase = (lanes + (g * 16)) * _D
            cbase = a_v[pl.ds(g * 16, 16)] * _D

            def dim_blk(b, carry):
                dl0, p0, p1, p2, p3 = carry
                accs = [p0, p1, p2, p3]
                for u in range(_UNROLL):
                    dlu = (dl0 + u) & dmask
                    vx = plsc.load_gather(x_v, [xbase + dlu])
                    vc = plsc.load_gather(c_v, [cbase + dlu])
                    diff = vx - vc
                    accs[u % 4] = accs[u % 4] + diff * diff
                dl0 = (dl0 + _UNROLL) & dmask
                return (dl0, accs[0], accs[1], accs[2], accs[3])

            _, p0, p1, p2, p3 = lax.fori_loop(
                0, _D // _UNROLL, dim_blk,
                (lanes, fzeros, fzeros, fzeros, fzeros))
            acc = acc + vmask * ((p0 + p1) + (p2 + p3))
        return acc

    start(0, bufs[0])

    def slot_pair(t, acc):
        s0 = t * 2
        start(s0 + 1, bufs[1])
        acc = process(s0, bufs[0], acc)
        start(s0 + 2, bufs[0])
        acc = process(s0 + 1, bufs[1], acc)
        return acc

    acc = lax.fori_loop(0, _SLOTS // 2, slot_pair, fzeros)
    pltpu.make_async_copy(x_hbm.at[pl.ds(0, _T * _D)], x_v0, sx0).wait()
    pltpu.make_async_copy(a_hbm.at[pl.ds(0, _T)], a_v0, sa0).wait()

    p_v[...] = acc
    pltpu.sync_copy(p_v, out_hbm.at[wid])


def _tc_loss_block(x_ref, a_ref, c_ref, out_ref):
    i = pl.program_id(0)
    x = x_ref[...]
    a = a_ref[0]
    c = c_ref[...]

    xs = jnp.sum(x * x)
    scores = jax.lax.dot_general(
        c, x, (((1,), (1,)), ((), ())), preferred_element_type=jnp.float32)
    cn = jnp.sum(c * c, axis=1, keepdims=True)
    m = cn - 2.0 * scores
    row = jax.lax.broadcasted_iota(jnp.int32, (_K, _B_TC), 0)
    oh = row == a
    s = jax.lax.broadcast(xs + jnp.sum(jnp.where(oh, m, 0.0)), (1, 1))

    @pl.when(i == 0)
    def _():
        out_ref[...] = s

    @pl.when(i != 0)
    def _():
        out_ref[...] += s


def _tc_loss(x, a3, c):
    return pl.pallas_call(
        _tc_loss_block,
        grid=(_NB_TC,),
        in_specs=[
            pl.BlockSpec((_B_TC, _D), lambda i: (i, 0)),
            pl.BlockSpec((1, 1, _B_TC), lambda i: (i, 0, 0)),
            pl.BlockSpec((_K, _D), lambda i: (0, 0)),
        ],
        out_specs=pl.BlockSpec((1, 1), lambda i: (0, 0)),
        out_shape=jax.ShapeDtypeStruct((1, 1), jnp.float32),
    )(x, a3, c)


def kernel(x, cluster_assignments, cluster_centers):
    a3 = cluster_assignments.reshape(_N // _B_TC, 1, _B_TC)
    sc_partials = _sc_loss(x.reshape(-1), cluster_assignments,
                           cluster_centers.reshape(-1))
    tc_part = _tc_loss(x, a3, cluster_centers)
    return tc_part[0, 0] + jnp.sum(sc_partials)
